# Initial kernel scaffold; baseline (speedup 1.0000x reference)
import jax
import jax.numpy as jnp
from jax.experimental import pallas as pl


def kernel(x, edge_index, W, b, gamma, beta):
    raise NotImplementedError("write your pallas kernel here")


# speedup vs baseline: 1.3523x; 1.3523x over previous
import functools

import jax
import jax.numpy as jnp
from jax import lax
from jax.experimental import pallas as pl
from jax.experimental.pallas import tpu as pltpu
from jax.experimental.pallas import tpu_sc as plsc

N = 10000
K = 32
C = 128
EPS = 1e-5

NW = 32
CN = 4
CHUNKS = 80
NPW = CN * CHUNKS
N_PAD = NW * NPW
L = 16


def _sc_gather_max(x2d, idx3d):
  mesh = plsc.VectorSubcoreMesh(core_axis_name="c", subcore_axis_name="s")

  def body(x_hbm, idx_hbm, out_hbm, idx_v, rows_v, stage_v, sem0, sem1):
    wid = lax.axis_index("s") * 2 + lax.axis_index("c")
    node_base = wid * NPW

    pltpu.sync_copy(idx_hbm.at[wid], idx_v)

    def node_max(slot, nd):
      base = nd * K
      init = tuple(rows_v[slot, base, pl.ds(l * L, L)] for l in range(8))

      def jbody(j, acc):
        return tuple(
            jnp.maximum(acc[l], rows_v[slot, base + j, pl.ds(l * L, L)])
            for l in range(8))

      acc = lax.fori_loop(1, K, jbody, init)
      for l in range(8):
        stage_v[nd, pl.ds(l * L, L)] = acc[l]

    def compute(chunk, slot):
      for nd in range(CN):
        node_max(slot, nd)
      pltpu.sync_copy(stage_v, out_hbm.at[pl.ds(node_base + chunk * CN, CN)])

    pltpu.async_copy(x_hbm.at[idx_v.at[0]], rows_v.at[0], sem0)

    def outer(c0, carry):
      nxt1 = pltpu.async_copy(x_hbm.at[idx_v.at[c0 + 1]], rows_v.at[1], sem1)
      pltpu.make_async_copy(x_hbm.at[idx_v.at[c0]], rows_v.at[0], sem0).wait()
      compute(c0, 0)

      @pl.when(c0 + 2 < CHUNKS)
      def _():
        pltpu.async_copy(x_hbm.at[idx_v.at[c0 + 2]], rows_v.at[0], sem0)

      pltpu.make_async_copy(
          x_hbm.at[idx_v.at[c0 + 1]], rows_v.at[1], sem1).wait()
      compute(c0 + 1, 1)
      return carry

    lax.fori_loop(0, CHUNKS // 2, lambda i, c: outer(i * 2, c), 0)

  f = pl.kernel(
      body,
      out_type=jax.ShapeDtypeStruct((N_PAD, C), jnp.float32),
      mesh=mesh,
      scratch_types=[
          pltpu.VMEM((CHUNKS, CN * K), jnp.int32),
          pltpu.VMEM((2, CN * K, C), jnp.float32),
          pltpu.VMEM((CN, C), jnp.float32),
          pltpu.SemaphoreType.DMA,
          pltpu.SemaphoreType.DMA,
      ],
  )
  return f(x2d, idx3d)


def _erf(v):
  return lax.erf(v)


def _tc_body(x_ref, m_ref, w1_ref, w2_ref, b_ref, g_ref, bt_ref, o_ref):
  x = x_ref[...]
  d = m_ref[...] - x
  h = lax.dot(x, w1_ref[...], precision=lax.Precision.HIGHEST,
              preferred_element_type=jnp.float32)
  h = h + lax.dot(d, w2_ref[...], precision=lax.Precision.HIGHEST,
                  preferred_element_type=jnp.float32)
  h = h + b_ref[...]
  mean = jnp.mean(h, axis=0, keepdims=True)
  cen = h - mean
  var = jnp.mean(cen * cen, axis=0, keepdims=True)
  hn = cen / jnp.sqrt(var + EPS)
  y = hn * g_ref[...] + bt_ref[...]
  o_ref[...] = 0.5 * y * (1.0 + _erf(y * 0.7071067811865476))


def kernel(x, edge_index, W, b, gamma, beta):
  Bb, Nn, Cc = x.shape
  x2d = x.reshape(Nn, Cc)
  idx = edge_index.reshape(Nn, K).astype(jnp.int32)
  idx = jnp.pad(idx, ((0, N_PAD - Nn), (0, 0)))
  idx3d = idx.reshape(NW, CHUNKS, CN * K)

  m = _sc_gather_max(x2d, idx3d)[:Nn]

  w1t = W[:, :Cc].T
  w2t = W[:, Cc:].T
  out = pl.pallas_call(
      _tc_body,
      out_shape=jax.ShapeDtypeStruct((Nn, W.shape[0]), jnp.float32),
  )(x2d, m, w1t, w2t, b.reshape(1, -1), gamma.reshape(1, -1),
    beta.reshape(1, -1))
  return out.reshape(Bb, Nn, -1)


# speedup vs baseline: 4.5847x; 3.3904x over previous
import functools

import jax
import jax.numpy as jnp
from jax import lax
from jax.experimental import pallas as pl
from jax.experimental.pallas import tpu as pltpu
from jax.experimental.pallas import tpu_sc as plsc

N = 10000
K = 32
C = 128
EPS = 1e-5

NW = 32
CN = 4
CHUNKS = 80
NPW = CN * CHUNKS
N_PAD = NW * NPW
L = 16


def _sc_gather_max(x2d, idx3d):
  mesh = plsc.VectorSubcoreMesh(core_axis_name="c", subcore_axis_name="s")
  rows_stage = N_PAD // 16

  def body(x_hbm, idx_hbm, out_hbm, xs, idx_v, rows_v, stage_v, sems):
    sid = lax.axis_index("s")
    wid = sid * 2 + lax.axis_index("c")
    node_base = wid * NPW

    r0 = sid * rows_stage
    pltpu.sync_copy(x_hbm.at[pl.ds(r0, rows_stage)],
                    xs.at[pl.ds(r0, rows_stage)])

    pltpu.sync_copy(idx_hbm.at[wid], idx_v)
    plsc.subcore_barrier()

    G = 8

    def node_max(slot, nd):
      base = nd * K
      acc = [rows_v[slot, base, pl.ds(l * L, L)] for l in range(8)]
      for j in range(1, G):
        for l in range(8):
          acc[l] = jnp.maximum(acc[l], rows_v[slot, base + j, pl.ds(l * L, L)])

      def jbody(g, a):
        b2 = base + g * G
        a = list(a)
        for j in range(G):
          for l in range(8):
            a[l] = jnp.maximum(a[l], rows_v[slot, b2 + j, pl.ds(l * L, L)])
        return tuple(a)

      acc = lax.fori_loop(1, K // G, jbody, tuple(acc))
      for l in range(8):
        stage_v[nd, pl.ds(l * L, L)] = acc[l]

    def compute(chunk, slot):
      for nd in range(CN):
        node_max(slot, nd)
      pltpu.sync_copy(stage_v, out_hbm.at[pl.ds(node_base + chunk * CN, CN)])

    pltpu.async_copy(xs.at[idx_v.at[0]], rows_v.at[0], sems.at[0])

    def outer(c0, carry):
      pltpu.async_copy(xs.at[idx_v.at[c0 + 1]], rows_v.at[1], sems.at[1])
      pltpu.make_async_copy(
          xs.at[idx_v.at[c0]], rows_v.at[0], sems.at[0]).wait()
      compute(c0, 0)

      @pl.when(c0 + 2 < CHUNKS)
      def _():
        pltpu.async_copy(xs.at[idx_v.at[c0 + 2]], rows_v.at[0], sems.at[0])

      pltpu.make_async_copy(
          xs.at[idx_v.at[c0 + 1]], rows_v.at[1], sems.at[1]).wait()
      compute(c0 + 1, 1)
      return carry

    lax.fori_loop(0, CHUNKS // 2, lambda i, c: outer(i * 2, c), 0)

  f = pl.kernel(
      body,
      out_type=jax.ShapeDtypeStruct((N_PAD, C), jnp.float32),
      mesh=mesh,
      scratch_types=[
          pltpu.VMEM_SHARED((N_PAD, C), jnp.float32),
          pltpu.VMEM((CHUNKS, CN * K), jnp.int32),
          pltpu.VMEM((2, CN * K, C), jnp.float32),
          pltpu.VMEM((CN, C), jnp.float32),
          pltpu.SemaphoreType.DMA((2,)),
      ],
  )
  return f(x2d, idx3d)


def _erf(v):
  return lax.erf(v)


def _tc_body(x_ref, m_ref, w1_ref, w2_ref, b_ref, g_ref, bt_ref, o_ref):
  x = x_ref[...]
  d = m_ref[...] - x
  h = lax.dot(x, w1_ref[...], precision=lax.Precision.HIGHEST,
              preferred_element_type=jnp.float32)
  h = h + lax.dot(d, w2_ref[...], precision=lax.Precision.HIGHEST,
                  preferred_element_type=jnp.float32)
  h = h + b_ref[...]
  mean = jnp.mean(h, axis=0, keepdims=True)
  cen = h - mean
  var = jnp.mean(cen * cen, axis=0, keepdims=True)
  hn = cen / jnp.sqrt(var + EPS)
  y = hn * g_ref[...] + bt_ref[...]
  o_ref[...] = 0.5 * y * (1.0 + _erf(y * 0.7071067811865476))


def kernel(x, edge_index, W, b, gamma, beta):
  Bb, Nn, Cc = x.shape
  x2d = x.reshape(Nn, Cc)
  x2d_pad = jnp.pad(x2d, ((0, N_PAD - Nn), (0, 0)))
  idx = edge_index.reshape(Nn, K).astype(jnp.int32)
  idx = jnp.pad(idx, ((0, N_PAD - Nn), (0, 0)))
  idx3d = idx.reshape(NW, CHUNKS, CN * K)

  m = _sc_gather_max(x2d_pad, idx3d)[:Nn]

  w1t = W[:, :Cc].T
  w2t = W[:, Cc:].T
  out = pl.pallas_call(
      _tc_body,
      out_shape=jax.ShapeDtypeStruct((Nn, W.shape[0]), jnp.float32),
  )(x2d, m, w1t, w2t, b.reshape(1, -1), gamma.reshape(1, -1),
    beta.reshape(1, -1))
  return out.reshape(Bb, Nn, -1)


# speedup vs baseline: 4.8759x; 1.0635x over previous
import functools

import jax
import jax.numpy as jnp
from jax import lax
from jax.experimental import pallas as pl
from jax.experimental.pallas import tpu as pltpu
from jax.experimental.pallas import tpu_sc as plsc

N = 10000
K = 32
C = 128
EPS = 1e-5

NW = 32
CN = 4
CHUNKS = 80
NPW = CN * CHUNKS
N_PAD = NW * NPW
L = 16


def _sc_gather_max(x2d, idx3d):
  mesh = plsc.VectorSubcoreMesh(core_axis_name="c", subcore_axis_name="s")
  rows_stage = (N // 16) // 8 * 8
  tail0 = 16 * rows_stage
  tail = N - tail0

  def body(x_hbm, idx_hbm, out_hbm, xs, idx_v, rows_v, stage_v, sems):
    sid = lax.axis_index("s")
    wid = sid * 2 + lax.axis_index("c")
    node_base = wid * NPW

    r0 = sid * rows_stage
    pltpu.sync_copy(x_hbm.at[pl.ds(r0, rows_stage)],
                    xs.at[pl.ds(r0, rows_stage)])

    @pl.when(sid == 0)
    def _():
      pltpu.sync_copy(x_hbm.at[pl.ds(tail0, tail)], xs.at[pl.ds(tail0, tail)])

    pltpu.sync_copy(idx_hbm.at[wid], idx_v)
    plsc.subcore_barrier()

    G = 8

    def node_max(slot, nd):
      base = nd * K
      acc = [rows_v[slot, base, pl.ds(l * L, L)] for l in range(8)]
      for j in range(1, G):
        for l in range(8):
          acc[l] = jnp.maximum(acc[l], rows_v[slot, base + j, pl.ds(l * L, L)])

      def jbody(g, a):
        b2 = base + g * G
        a = list(a)
        for j in range(G):
          for l in range(8):
            a[l] = jnp.maximum(a[l], rows_v[slot, b2 + j, pl.ds(l * L, L)])
        return tuple(a)

      acc = lax.fori_loop(1, K // G, jbody, tuple(acc))
      for l in range(8):
        stage_v[nd, pl.ds(l * L, L)] = acc[l]

    def compute(chunk, slot):
      for nd in range(CN):
        node_max(slot, nd)
      pltpu.sync_copy(stage_v, out_hbm.at[pl.ds(node_base + chunk * CN, CN)])

    pltpu.async_copy(xs.at[idx_v.at[0]], rows_v.at[0], sems.at[0])

    def outer(c0, carry):
      pltpu.async_copy(xs.at[idx_v.at[c0 + 1]], rows_v.at[1], sems.at[1])
      pltpu.make_async_copy(
          xs.at[idx_v.at[c0]], rows_v.at[0], sems.at[0]).wait()
      compute(c0, 0)

      @pl.when(c0 + 2 < CHUNKS)
      def _():
        pltpu.async_copy(xs.at[idx_v.at[c0 + 2]], rows_v.at[0], sems.at[0])

      pltpu.make_async_copy(
          xs.at[idx_v.at[c0 + 1]], rows_v.at[1], sems.at[1]).wait()
      compute(c0 + 1, 1)
      return carry

    lax.fori_loop(0, CHUNKS // 2, lambda i, c: outer(i * 2, c), 0)

  f = pl.kernel(
      body,
      out_type=jax.ShapeDtypeStruct((N_PAD, C), jnp.float32),
      mesh=mesh,
      scratch_types=[
          pltpu.VMEM_SHARED((N, C), jnp.float32),
          pltpu.VMEM((CHUNKS, CN * K), jnp.int32),
          pltpu.VMEM((2, CN * K, C), jnp.float32),
          pltpu.VMEM((CN, C), jnp.float32),
          pltpu.SemaphoreType.DMA((2,)),
      ],
  )
  return f(x2d, idx3d)


def _erf(v):
  return lax.erf(v)


def _tc_body(x_ref, m_ref, w1_ref, w2_ref, b_ref, g_ref, bt_ref, o_ref):
  x = x_ref[...]
  d = m_ref[0:N, :] - x
  h = lax.dot(x, w1_ref[...], precision=lax.Precision.HIGHEST,
              preferred_element_type=jnp.float32)
  h = h + lax.dot(d, w2_ref[...], precision=lax.Precision.HIGHEST,
                  preferred_element_type=jnp.float32)
  h = h + b_ref[...]
  mean = jnp.mean(h, axis=0, keepdims=True)
  cen = h - mean
  var = jnp.mean(cen * cen, axis=0, keepdims=True)
  hn = cen / jnp.sqrt(var + EPS)
  y = hn * g_ref[...] + bt_ref[...]
  o_ref[...] = 0.5 * y * (1.0 + _erf(y * 0.7071067811865476))


def kernel(x, edge_index, W, b, gamma, beta):
  Bb, Nn, Cc = x.shape
  x2d = x.reshape(Nn, Cc)
  idx = edge_index.reshape(Nn, K).astype(jnp.int32)
  idx = jnp.pad(idx, ((0, N_PAD - Nn), (0, 0)))
  idx3d = idx.reshape(NW, CHUNKS, CN * K)

  m = _sc_gather_max(x2d, idx3d)

  w1t = W[:, :Cc].T
  w2t = W[:, Cc:].T
  out = pl.pallas_call(
      _tc_body,
      out_shape=jax.ShapeDtypeStruct((Nn, W.shape[0]), jnp.float32),
  )(x2d, m, w1t, w2t, b.reshape(1, -1), gamma.reshape(1, -1),
    beta.reshape(1, -1))
  return out.reshape(Bb, Nn, -1)
